# baseline (speedup 1.0000x reference)
import functools

import jax
import jax.numpy as jnp
from jax import lax
from jax.experimental import pallas as pl
from jax.experimental.pallas import tpu as pltpu
from jax.experimental.pallas import tpu_sc as plsc

B, N, C, H, W, R = 4, 8192, 64, 128, 128, 8
BN = B * N
NW = 32
P_TILE = BN // NW
CH = 128
NCH = P_TILE // CH
ROWS_PER_BATCH = H * W * R


def _sc_body(xs_h, ys_h, as_h, lw_h, obs_h, table_h, out_h,
             x_v, y_v, a_v, lw_v, obs_v, w_v,
             idx_refs, wg_refs, row_refs, sem):
    cid = lax.axis_index("c")
    sid = lax.axis_index("s")
    wid = sid * 2 + cid
    b = wid // (NW // B)
    base_row = b * ROWS_PER_BATCH
    pstart = wid * P_TILE

    pltpu.sync_copy(obs_h.at[pl.ds(b * C, C)], obs_v)

    @pl.loop(0, NCH)
    def _chunk(ci):
        base = pstart + ci * CH
        pltpu.sync_copy(xs_h.at[pl.ds(base, CH)], x_v)
        pltpu.sync_copy(ys_h.at[pl.ds(base, CH)], y_v)
        pltpu.sync_copy(as_h.at[pl.ds(base, CH)], a_v)
        pltpu.sync_copy(lw_h.at[pl.ds(base, CH)], lw_v)

        @pl.loop(0, CH // 16)
        def _grp(g):
            sl = pl.ds(g * 16, 16)
            x = x_v[sl]
            y = y_v[sl]
            ang = a_v[sl]
            t = ang * (1.0 / 360.0)
            t = t - t.astype(jnp.int32).astype(jnp.float32)
            a = t * 8.0
            x0 = jnp.minimum(x.astype(jnp.int32), W - 2)
            y0 = jnp.minimum(y.astype(jnp.int32), H - 2)
            a0 = jnp.minimum(a.astype(jnp.int32), R - 1)
            fx = x - x0.astype(jnp.float32)
            fy = y - y0.astype(jnp.float32)
            fa = a - a0.astype(jnp.float32)
            gx = 1.0 - fx
            gy = 1.0 - fy
            ga = 1.0 - fa
            a1 = a0 + 1
            a1 = jnp.where(a1 == R, 0, a1)
            r00 = base_row + (y0 * W + x0) * R
            r01 = r00 + R
            r10 = r00 + W * R
            r11 = r10 + R
            idx_refs[0][sl] = r00 + a0
            idx_refs[1][sl] = r00 + a1
            idx_refs[2][sl] = r01 + a0
            idx_refs[3][sl] = r01 + a1
            idx_refs[4][sl] = r10 + a0
            idx_refs[5][sl] = r10 + a1
            idx_refs[6][sl] = r11 + a0
            idx_refs[7][sl] = r11 + a1
            wg_refs[0][sl] = gy * gx * ga
            wg_refs[1][sl] = gy * gx * fa
            wg_refs[2][sl] = gy * fx * ga
            wg_refs[3][sl] = gy * fx * fa
            wg_refs[4][sl] = fy * gx * ga
            wg_refs[5][sl] = fy * gx * fa
            wg_refs[6][sl] = fy * fx * ga
            wg_refs[7][sl] = fy * fx * fa

        copies = [pltpu.async_copy(table_h.at[idx_refs[j]], row_refs[j], sem)
                  for j in range(8)]
        for cp in copies:
            cp.wait()


        @pl.loop(0, CH // 16)
        def _grp2(g):
            sl = pl.ds(g * 16, 16)
            pidx = lax.iota(jnp.int32, 16) + g * 16
            ws = [wg_refs[j][sl] for j in range(8)]

            def cbody(cc, d2):
                cs = jnp.full((16,), cc, jnp.int32)
                m = plsc.load_gather(row_refs[0], [pidx, cs]) * ws[0]
                for j in range(1, 8):
                    m = m + plsc.load_gather(row_refs[j], [pidx, cs]) * ws[j]
                o = plsc.load_gather(obs_v, [cs])
                u = m - o
                return d2 + u * u

            d2 = lax.fori_loop(0, C, cbody, jnp.zeros((16,), jnp.float32),
                               unroll=4)
            w_v[sl] = jnp.exp(lw_v[sl] - d2 * (1.0 / 128.0))

        pltpu.sync_copy(w_v, out_h.at[pl.ds(base, CH)])


def _sc_weights(xs, ys, angs, lws, obs_flat, table):
    mesh = plsc.VectorSubcoreMesh(core_axis_name="c", subcore_axis_name="s",
                                  num_cores=2, num_subcores=16)
    fn = pl.kernel(
        functools.partial(_wrap_body),
        out_type=jax.ShapeDtypeStruct((BN,), jnp.float32),
        mesh=mesh,
        scratch_types=dict(
            x_v=pltpu.VMEM((CH,), jnp.float32),
            y_v=pltpu.VMEM((CH,), jnp.float32),
            a_v=pltpu.VMEM((CH,), jnp.float32),
            lw_v=pltpu.VMEM((CH,), jnp.float32),
            obs_v=pltpu.VMEM((C,), jnp.float32),
            w_v=pltpu.VMEM((CH,), jnp.float32),
            idx_refs=[pltpu.VMEM((CH,), jnp.int32) for _ in range(8)],
            wg_refs=[pltpu.VMEM((CH,), jnp.float32) for _ in range(8)],
            row_refs=[pltpu.VMEM((CH, C), jnp.float32) for _ in range(8)],
            sem=pltpu.SemaphoreType.DMA,
        ),
        compiler_params=pltpu.CompilerParams(needs_layout_passes=False, use_tc_tiling_on_sc=False),
    )
    return fn(xs, ys, angs, lws, obs_flat, table)


def _wrap_body(xs_h, ys_h, as_h, lw_h, obs_h, table_h, out_h, *,
               x_v, y_v, a_v, lw_v, obs_v, w_v, idx_refs, wg_refs, row_refs,
               sem):
    _sc_body(xs_h, ys_h, as_h, lw_h, obs_h, table_h, out_h,
             x_v, y_v, a_v, lw_v, obs_v, w_v, idx_refs, wg_refs, row_refs,
             sem)


def _norm_body(w_ref, o_ref):
    w = w_ref[...]
    o_ref[...] = w / jnp.sum(w, axis=1, keepdims=True)


def kernel(particles, encoded_global_map, encoded_observations,
           unnormalized_resampled_particle_log_weights):
    table = jnp.moveaxis(encoded_global_map, 1, -1).reshape(B * H * W * R, C)
    xs = particles[..., 0].reshape(BN)
    ys = particles[..., 1].reshape(BN)
    angs = particles[..., 2].reshape(BN)
    lws = unnormalized_resampled_particle_log_weights.reshape(BN)
    obs_flat = encoded_observations.reshape(B * C)
    w_un = _sc_weights(xs, ys, angs, lws, obs_flat, table).reshape(B, N)
    return pl.pallas_call(
        _norm_body,
        out_shape=jax.ShapeDtypeStruct((B, N), jnp.float32),
    )(w_un)
